# baseline (speedup 1.0000x reference)
import dataclasses

import jax
import jax.numpy as jnp
from jax import lax
from jax.experimental import pallas as pl
from jax.experimental.pallas import tpu as pltpu
from jax.experimental.pallas import tpu_sc as plsc

N = 10000
F = 128
FH = 64
NCLS = 16

NC = 2
NS = 16
L = 16
NW = NC * NS

GARB = N
NACC = N + 112
RPT = NACC // NS
SRT = N // NS

CHUNK = 128
SWEEP = 80
E = 320000
EPAD = 327680
NCH = EPAD // CHUNK

_HIGH = lax.Precision.HIGHEST


def _sc_params(tc_tiling=True):
    cp = pltpu.CompilerParams()
    if "needs_layout_passes" in pltpu.CompilerParams.__dataclass_fields__:
        cp = dataclasses.replace(cp, needs_layout_passes=False)
    if not tc_tiling:
        cp = dataclasses.replace(cp, use_tc_tiling_on_sc=False)
    return cp


def _dot(a, b):
    return lax.dot_general(a, b, (((1,), (0,)), ((), ())),
                           precision=_HIGH, preferred_element_type=jnp.float32)


def _zero_1d(ref, nwords):
    @pl.loop(0, nwords // L)
    def _(i):
        ref[pl.ds(i * L, L)] = jnp.zeros((L,), jnp.float32)


def _make_hist_body(nlists):
    def body(*refs):
        d_hbms = refs[:nlists]
        out_hbm = refs[nlists]
        dch, hacc, sem = refs[nlists + 1:]

        cid = lax.axis_index("c")
        sid = lax.axis_index("s")
        wid = sid * NC + cid

        _zero_1d(hacc, NACC)
        npw = NCH // NW
        ones = jnp.ones((L,), jnp.float32)

        for d_hbm in d_hbms:
            pltpu.async_copy(d_hbm.at[pl.ds(wid * npw, npw)], dch, sem).wait()

            @pl.loop(0, npw)
            def _(g):
                for j in range(CHUNK // L):
                    didv = dch[g, pl.ds(j * L, L)]
                    plsc.addupdate_scatter(hacc, [didv], ones)

        pltpu.sync_copy(hacc, out_hbm.at[wid])

    return body


def _hist(arrs):
    mesh = plsc.VectorSubcoreMesh(core_axis_name="c", subcore_axis_name="s")
    k = pl.kernel(
        _make_hist_body(len(arrs)),
        out_type=jax.ShapeDtypeStruct((NW, NACC), jnp.float32),
        mesh=mesh,
        name=f"schist{len(arrs)}",
        compiler_params=_sc_params(),
        scratch_types=[
            pltpu.VMEM((NCH // NW, CHUNK), jnp.int32),
            pltpu.VMEM((NACC,), jnp.float32),
            pltpu.SemaphoreType.DMA,
        ],
    )
    return k(*arrs)


def _make_agg_body(d, nchs, split_features):
    nl = len(nchs)
    ninp = 1

    def body(*refs):
        h_hbms = refs[:ninp]
        edges = refs[ninp:ninp + 2 * nl]
        out_hbm = refs[ninp + 2 * nl]
        (sidx2, didx2, r0, r1, tbl, acc, isem, a0, a1) = refs[ninp + 1 + 2 * nl:]
        rows = (r0, r1)
        ssem = (a0, a1)

        cid = lax.axis_index("c")
        sid = lax.axis_index("s")
        wid = sid * NC + cid

        srow = sid * SRT
        src_tbl = h_hbms[0].at[cid] if split_features else h_hbms[0]
        pltpu.sync_copy(src_tbl.at[pl.ds(srow, SRT)], tbl.at[pl.ds(srow, SRT)])

        @pl.loop(0, CHUNK)
        def _(r):
            @pl.loop(0, d // L if d >= L else 1)
            def _(k):
                r0[r, pl.ds(k * L, L)] = jnp.zeros((L,), jnp.float32)

        tb = sid * RPT

        @pl.loop(0, RPT // CHUNK)
        def _(i):
            pltpu.sync_copy(r0, acc.at[pl.ds(tb + i * CHUNK, CHUNK)])

        rem = RPT % CHUNK
        if rem:
            pltpu.sync_copy(r0.at[pl.ds(0, rem)],
                            acc.at[pl.ds(tb + (RPT // CHUNK) * CHUNK, rem)])
        plsc.subcore_barrier()

        for li in range(nl):
            s_hbm = edges[2 * li]
            dd_hbm = edges[2 * li + 1]
            if split_features:
                npt = nchs[li] // NS
                base0 = sid * npt
            else:
                npt = nchs[li] // NW
                base0 = wid * npt
            nsw = npt // SWEEP

            for sw in range(nsw):
                base = base0 + sw * SWEEP
                pltpu.async_copy(s_hbm.at[pl.ds(base, SWEEP)], sidx2, isem)
                pltpu.async_copy(dd_hbm.at[pl.ds(base, SWEEP)], didx2, isem)
                pltpu.make_async_copy(s_hbm.at[pl.ds(base, SWEEP)], sidx2, isem).wait()
                pltpu.make_async_copy(dd_hbm.at[pl.ds(base, SWEEP)], didx2, isem).wait()

                def wait_scat(g, b):
                    pltpu.make_async_copy(rows[b], acc.at[didx2.at[g]], ssem[b]).wait()

                @pl.loop(0, SWEEP // 2)
                def _(i):
                    for b in (0, 1):
                        g = i * 2 + b

                        @pl.when(g >= 2)
                        def _():
                            wait_scat(g - 2, b)

                        gd = pltpu.async_copy(tbl.at[sidx2.at[g]], rows[b], ssem[b])
                        gd.wait()
                        pltpu.async_copy(rows[b], acc.at[didx2.at[g]], ssem[b], add=True)

                wait_scat(SWEEP - 2, 0)
                wait_scat(SWEEP - 1, 1)

        plsc.subcore_barrier()
        pltpu.sync_copy(acc.at[pl.ds(tb, RPT)], out_hbm.at[cid].at[pl.ds(tb, RPT)])

    return body


def _agg(h, pairs):
    split = h.ndim == 3
    dst = int(h.shape[-1])
    d = dst * (2 if split else 1)
    nchs = tuple(int(s.shape[0]) for s, _ in pairs)
    mesh = plsc.VectorSubcoreMesh(core_axis_name="c", subcore_axis_name="s")
    k = pl.kernel(
        _make_agg_body(dst, nchs, split),
        out_type=jax.ShapeDtypeStruct((NC, NACC, dst), jnp.float32),
        mesh=mesh,
        name=f"scagg{d}_{len(nchs)}",
        compiler_params=_sc_params(tc_tiling=False),
        scratch_types=[
            pltpu.VMEM((SWEEP, CHUNK), jnp.int32),
            pltpu.VMEM((SWEEP, CHUNK), jnp.int32),
            pltpu.VMEM((CHUNK, dst), jnp.float32),
            pltpu.VMEM((CHUNK, dst), jnp.float32),
            pltpu.VMEM_SHARED((N, dst), jnp.float32),
            pltpu.VMEM_SHARED((NACC, dst), jnp.float32),
            pltpu.SemaphoreType.DMA,
            pltpu.SemaphoreType.DMA,
            pltpu.SemaphoreType.DMA,
        ],
    )
    args = [h]
    for s, dd in pairs:
        args += [s, dd]
    return k(*args)


def _dec_body(zs_hbm, u_hbm, v_hbm, out_hbm,
              uch2, vch2, zu0, zu1, zv0, zv1, p0, p1, ztbl,
              isem, g0, g1, o0, o1):
    zu = (zu0, zu1)
    zv = (zv0, zv1)
    pch = (p0, p1)
    gsem = (g0, g1)
    osem = (o0, o1)

    cid = lax.axis_index("c")
    sid = lax.axis_index("s")

    srow = sid * SRT
    pltpu.sync_copy(zs_hbm.at[cid].at[pl.ds(srow, SRT)], ztbl.at[pl.ds(srow, SRT)])
    plsc.subcore_barrier()

    npt = NCH // NS
    nsw = npt // SWEEP
    lane = lax.iota(jnp.int32, L)

    for sw in range(nsw):
        base = sid * npt + sw * SWEEP
        pltpu.async_copy(u_hbm.at[pl.ds(base, SWEEP)], uch2, isem)
        pltpu.async_copy(v_hbm.at[pl.ds(base, SWEEP)], vch2, isem)
        pltpu.make_async_copy(u_hbm.at[pl.ds(base, SWEEP)], uch2, isem).wait()
        pltpu.make_async_copy(v_hbm.at[pl.ds(base, SWEEP)], vch2, isem).wait()

        def issue_gather(g, b):
            pltpu.async_copy(ztbl.at[uch2.at[g]], zu[b], gsem[b])
            pltpu.async_copy(ztbl.at[vch2.at[g]], zv[b], gsem[b])

        def wait_gather(g, b):
            pltpu.make_async_copy(ztbl.at[uch2.at[g]], zu[b], gsem[b]).wait()
            pltpu.make_async_copy(ztbl.at[vch2.at[g]], zv[b], gsem[b]).wait()

        def issue_out(g, b, base=base):
            pltpu.async_copy(pch[b], out_hbm.at[cid].at[base + g], osem[b])

        def wait_out(g, b, base=base):
            pltpu.make_async_copy(pch[b], out_hbm.at[cid].at[base + g], osem[b]).wait()

        def compute(g, b):
            for j in range(CHUNK // L):
                rows_i = lane + (j * L)

                def kblock(i, acc):
                    k0 = i * 8
                    ps = []
                    for kk in range(8):
                        cols = jnp.full((L,), k0 + kk, jnp.int32)
                        a = plsc.load_gather(zu[b], [rows_i, cols])
                        c = plsc.load_gather(zv[b], [rows_i, cols])
                        ps.append(a * c)
                    s = (((ps[0] + ps[1]) + (ps[2] + ps[3]))
                         + ((ps[4] + ps[5]) + (ps[6] + ps[7])))
                    return acc + s

                dacc = lax.fori_loop(0, FH // 8, kblock,
                                     jnp.zeros((L,), jnp.float32))
                pch[b][pl.ds(j * L, L)] = dacc

        issue_gather(0, 0)

        @pl.loop(0, SWEEP // 2)
        def _(i):
            for b in (0, 1):
                g = i * 2 + b
                b2 = 1 - b

                @pl.when(g + 1 < SWEEP)
                def _():
                    issue_gather(g + 1, b2)

                wait_gather(g, b)

                @pl.when(g >= 2)
                def _():
                    wait_out(g - 2, b)

                compute(g, b)
                issue_out(g, b)

        wait_out(SWEEP - 2, 0)
        wait_out(SWEEP - 1, 1)


def _decode(zs, u_pad, v_pad):
    mesh = plsc.VectorSubcoreMesh(core_axis_name="c", subcore_axis_name="s")
    k = pl.kernel(
        _dec_body,
        name="scdec",
        out_type=jax.ShapeDtypeStruct((NC, NCH, CHUNK), jnp.float32),
        mesh=mesh,
        compiler_params=_sc_params(tc_tiling=False),
        scratch_types=(
            [pltpu.VMEM((SWEEP, CHUNK), jnp.int32)] * 2
            + [pltpu.VMEM((CHUNK, FH), jnp.float32)] * 4
            + [pltpu.VMEM((CHUNK,), jnp.float32)] * 2
            + [pltpu.VMEM_SHARED((N, FH), jnp.float32)]
            + [pltpu.SemaphoreType.DMA] * 5
        ),
    )
    return k(zs, u_pad, v_pad)


def _tc(fn, out_shape, *args):
    return pl.pallas_call(fn, out_shape=out_shape)(*args)


def _f32(shape):
    return jax.ShapeDtypeStruct(shape, jnp.float32)


def _cat(p_r):
    return jnp.concatenate([p_r[0, :N, :], p_r[1, :N, :]], axis=1)


def _tcsplit(x_r, s_r):
    x = x_r[...]
    s_r[0, :, :] = x[:, :FH]
    s_r[1, :, :] = x[:, FH:]


def _split(x):
    return _tc(_tcsplit, _f32((NC, N, FH)), x)


def _tc1(x_r, w1_r, w1s_r, h1_r, hs_r):
    x = x_r[...]
    h1_r[...] = _dot(x, w1_r[...])
    hs_r[...] = _dot(x, w1s_r[...])


def _tc2(h1_r, hp_r, h1p_r, dis_r, deg_r):
    degv = 1.0 + jnp.sum(hp_r[...][:, :N], axis=0)
    dis = lax.rsqrt(degv)[:, None]
    deg_r[...] = degv[:, None]
    dis_r[...] = dis
    h1p_r[...] = h1_r[...] * dis


def _tc3(p_r, h1p_r, dis_r, w2_r, b1_r, g2p_r):
    agg = _cat(p_r)
    dis = dis_r[...]
    x2 = jnp.maximum((agg + h1p_r[...]) * dis + b1_r[...], 0.0)
    g2p_r[...] = _dot(x2, w2_r[...]) * dis


def _tc4(p_r, g2p_r, dis_r, b2_r, z_r):
    agg = _cat(p_r)
    z_r[...] = (agg + g2p_r[...]) * dis_r[...] + b2_r[...]


def _tcmask(pd_r, cu_r, cv_r, dv_r, du_r, cnt_r):
    d = pd_r[0] + pd_r[1]
    row = lax.broadcasted_iota(jnp.int32, (NCH, CHUNK), 0)
    col = lax.broadcasted_iota(jnp.int32, (NCH, CHUNK), 1)
    gi = row * CHUNK + col
    sel = (d > 0.0) & (gi < E)
    dv_r[...] = jnp.where(sel, cv_r[...], GARB)
    du_r[...] = jnp.where(sel, cu_r[...], GARB)
    cnt_r[...] = jnp.sum(sel.astype(jnp.float32)).astype(jnp.int32).reshape(1, 1)


def _tc5(hp_r, deg_r, hs_r, hsp_r, diss_r):
    degs = deg_r[...] + jnp.sum(hp_r[...][:, :N], axis=0)[:, None]
    diss = lax.rsqrt(degs)
    diss_r[...] = diss
    hsp_r[...] = hs_r[...] * diss


def _tc6(p_r, hsp_r, diss_r, w2s_r, b1s_r, g4p_r):
    agg = _cat(p_r)
    diss = diss_r[...]
    h2 = jnp.maximum((agg + hsp_r[...]) * diss + b1s_r[...], 0.0)
    g4p_r[...] = _dot(h2, w2s_r[...]) * diss


def _tc7(p_r, g4p_r, diss_r, b2s_r, out_r):
    agg = p_r[0, :N, :] + p_r[1, :N, :]
    o = (agg + g4p_r[...]) * diss_r[...] + b2s_r[...]
    mx = jnp.max(o, axis=1, keepdims=True)
    ls = mx + jnp.log(jnp.sum(jnp.exp(o - mx), axis=1, keepdims=True))
    out_r[...] = o - ls


def kernel(x, masked_nodes, pos_edge_index, neg_edge_index, edge_index,
           W1, b1, W2, b2, W1s, b1s, W2s, b2s):
    del masked_nodes
    ei = edge_index.astype(jnp.int32)
    pe = pos_edge_index.astype(jnp.int32)
    ne = neg_edge_index.astype(jnp.int32)
    npad = EPAD - E
    zpad = jnp.zeros((npad,), jnp.int32)
    gpad = jnp.full((npad,), GARB, jnp.int32)
    es = jnp.concatenate([ei[0], zpad]).reshape(NCH, CHUNK)
    ed = jnp.concatenate([ei[1], gpad]).reshape(NCH, CHUNK)
    cu = jnp.concatenate([pe[0], ne[0], zpad]).reshape(NCH, CHUNK)
    cv = jnp.concatenate([pe[1], ne[1], zpad]).reshape(NCH, CHUNK)

    b1r = b1.reshape(1, F)
    b2r = b2.reshape(1, F)
    b1sr = b1s.reshape(1, F)
    b2sr = b2s.reshape(1, NCLS)

    histE = _hist([ed])
    h1, hs = _tc(_tc1, (_f32((N, F)), _f32((N, F))), x, W1, W1s)
    h1p, dis, deg = _tc(
        _tc2, (_f32((N, F)), _f32((N, 1)), _f32((N, 1))), h1, histE)

    p1 = _agg(_split(h1p), [(es, ed)])
    g2p = _tc(_tc3, _f32((N, F)), p1, h1p, dis, W2, b1r)

    p2 = _agg(_split(g2p), [(es, ed)])
    z = _tc(_tc4, _f32((N, F)), p2, g2p, dis, b2r)

    pdot = _decode(_split(z), cu, cv)
    dstv, dstu, cnt = _tc(
        _tcmask,
        (jax.ShapeDtypeStruct((NCH, CHUNK), jnp.int32),
         jax.ShapeDtypeStruct((NCH, CHUNK), jnp.int32),
         jax.ShapeDtypeStruct((1, 1), jnp.int32)),
        pdot, cu, cv)

    histM = _hist([dstv, dstu])
    hsp, diss = _tc(_tc5, (_f32((N, F)), _f32((N, 1))), histM, deg, hs)

    p3 = _agg(_split(hsp), [(es, ed), (cu, dstv), (cv, dstu)])
    g4p = _tc(_tc6, _f32((N, NCLS)), p3, hsp, diss, W2s, b1sr)

    p4 = _agg(g4p, [(es, ed), (cu, dstv), (cv, dstu)])
    logits = _tc(_tc7, _f32((N, NCLS)), p4, g4p, diss, b2sr)

    return (logits, z, cnt.reshape(()))


# speedup vs baseline: 1.9655x; 1.9655x over previous
import dataclasses

import jax
import jax.numpy as jnp
from jax import lax
from jax.experimental import pallas as pl
from jax.experimental.pallas import tpu as pltpu
from jax.experimental.pallas import tpu_sc as plsc

N = 10000
F = 128
FH = 64
NCLS = 16

NC = 2
NS = 16
L = 16
NW = NC * NS

GARB = N
NACC = N + 112
RPT = NACC // NS
SRT = N // NS

CHUNK = 128
SWEEP = 80
E = 320000
EPAD = 327680
NCH = EPAD // CHUNK

_HIGH = lax.Precision.HIGHEST


def _sc_params(tc_tiling=True):
    cp = pltpu.CompilerParams()
    if "needs_layout_passes" in pltpu.CompilerParams.__dataclass_fields__:
        cp = dataclasses.replace(cp, needs_layout_passes=False)
    if not tc_tiling:
        cp = dataclasses.replace(cp, use_tc_tiling_on_sc=False)
    return cp


def _dot(a, b):
    return lax.dot_general(a, b, (((1,), (0,)), ((), ())),
                           precision=_HIGH, preferred_element_type=jnp.float32)


def _zero_1d(ref, nwords):
    @pl.loop(0, nwords // L)
    def _(i):
        ref[pl.ds(i * L, L)] = jnp.zeros((L,), jnp.float32)


def _make_hist_body(nlists):
    def body(*refs):
        d_hbms = refs[:nlists]
        out_hbm = refs[nlists]
        dch, hacc, sem = refs[nlists + 1:]

        cid = lax.axis_index("c")
        sid = lax.axis_index("s")
        wid = sid * NC + cid

        _zero_1d(hacc, NACC)
        npw = NCH // NW
        ones = jnp.ones((L,), jnp.float32)

        for d_hbm in d_hbms:
            pltpu.async_copy(d_hbm.at[pl.ds(wid * npw, npw)], dch, sem).wait()

            @pl.loop(0, npw)
            def _(g):
                for j in range(CHUNK // L):
                    didv = dch[g, pl.ds(j * L, L)]
                    plsc.addupdate_scatter(hacc, [didv], ones)

        pltpu.sync_copy(hacc, out_hbm.at[wid])

    return body


def _hist(arrs):
    mesh = plsc.VectorSubcoreMesh(core_axis_name="c", subcore_axis_name="s")
    k = pl.kernel(
        _make_hist_body(len(arrs)),
        out_type=jax.ShapeDtypeStruct((NW, NACC), jnp.float32),
        mesh=mesh,
        name=f"schist{len(arrs)}",
        compiler_params=_sc_params(),
        scratch_types=[
            pltpu.VMEM((NCH // NW, CHUNK), jnp.int32),
            pltpu.VMEM((NACC,), jnp.float32),
            pltpu.SemaphoreType.DMA,
        ],
    )
    return k(*arrs)


def _make_agg_body(d, nchs, split_features):
    nl = len(nchs)
    ninp = 1

    def body(*refs):
        h_hbms = refs[:ninp]
        edges = refs[ninp:ninp + 2 * nl]
        out_hbm = refs[ninp + 2 * nl]
        (sidx2, didx2, r0, r1, tbl, acc, isem, a0, a1) = refs[ninp + 1 + 2 * nl:]
        rows = (r0, r1)
        ssem = (a0, a1)

        cid = lax.axis_index("c")
        sid = lax.axis_index("s")
        wid = sid * NC + cid

        srow = sid * SRT
        src_tbl = h_hbms[0].at[cid] if split_features else h_hbms[0]
        pltpu.sync_copy(src_tbl.at[pl.ds(srow, SRT)], tbl.at[pl.ds(srow, SRT)])

        @pl.loop(0, CHUNK)
        def _(r):
            @pl.loop(0, d // L if d >= L else 1)
            def _(k):
                r0[r, pl.ds(k * L, L)] = jnp.zeros((L,), jnp.float32)

        tb = sid * RPT

        @pl.loop(0, RPT // CHUNK)
        def _(i):
            pltpu.sync_copy(r0, acc.at[pl.ds(tb + i * CHUNK, CHUNK)])

        rem = RPT % CHUNK
        if rem:
            pltpu.sync_copy(r0.at[pl.ds(0, rem)],
                            acc.at[pl.ds(tb + (RPT // CHUNK) * CHUNK, rem)])
        plsc.subcore_barrier()

        for li in range(nl):
            s_hbm = edges[2 * li]
            dd_hbm = edges[2 * li + 1]
            if split_features:
                npt = nchs[li] // NS
                base0 = sid * npt
            else:
                npt = nchs[li] // NW
                base0 = wid * npt
            nsw = npt // SWEEP

            for sw in range(nsw):
                base = base0 + sw * SWEEP
                pltpu.async_copy(s_hbm.at[pl.ds(base, SWEEP)], sidx2, isem)
                pltpu.async_copy(dd_hbm.at[pl.ds(base, SWEEP)], didx2, isem)
                pltpu.make_async_copy(s_hbm.at[pl.ds(base, SWEEP)], sidx2, isem).wait()
                pltpu.make_async_copy(dd_hbm.at[pl.ds(base, SWEEP)], didx2, isem).wait()

                def wait_scat(g, b):
                    pltpu.make_async_copy(rows[b], acc.at[didx2.at[g]], ssem[b]).wait()

                @pl.loop(0, SWEEP // 2)
                def _(i):
                    for b in (0, 1):
                        g = i * 2 + b

                        @pl.when(g >= 2)
                        def _():
                            wait_scat(g - 2, b)

                        gd = pltpu.async_copy(tbl.at[sidx2.at[g]], rows[b], ssem[b])
                        gd.wait()
                        pltpu.async_copy(rows[b], acc.at[didx2.at[g]], ssem[b], add=True)

                wait_scat(SWEEP - 2, 0)
                wait_scat(SWEEP - 1, 1)

        plsc.subcore_barrier()
        pltpu.sync_copy(acc.at[pl.ds(tb, RPT)], out_hbm.at[cid].at[pl.ds(tb, RPT)])

    return body


def _agg(h, pairs):
    split = h.ndim == 3
    dst = int(h.shape[-1])
    d = dst * (2 if split else 1)
    nchs = tuple(int(s.shape[0]) for s, _ in pairs)
    mesh = plsc.VectorSubcoreMesh(core_axis_name="c", subcore_axis_name="s")
    k = pl.kernel(
        _make_agg_body(dst, nchs, split),
        out_type=jax.ShapeDtypeStruct((NC, NACC, dst), jnp.float32),
        mesh=mesh,
        name=f"scagg{d}_{len(nchs)}",
        compiler_params=_sc_params(tc_tiling=False),
        scratch_types=[
            pltpu.VMEM((SWEEP, CHUNK), jnp.int32),
            pltpu.VMEM((SWEEP, CHUNK), jnp.int32),
            pltpu.VMEM((CHUNK, dst), jnp.float32),
            pltpu.VMEM((CHUNK, dst), jnp.float32),
            pltpu.VMEM_SHARED((N, dst), jnp.float32),
            pltpu.VMEM_SHARED((NACC, dst), jnp.float32),
            pltpu.SemaphoreType.DMA,
            pltpu.SemaphoreType.DMA,
            pltpu.SemaphoreType.DMA,
        ],
    )
    args = [h]
    for s, dd in pairs:
        args += [s, dd]
    return k(*args)


def _dec_body(zs_hbm, u_hbm, v_hbm, out_hbm,
              uch2, vch2, zu0, zu1, zv0, zv1, p0, p1, ztbl,
              isem, g0, g1, o0, o1):
    zu = (zu0, zu1)
    zv = (zv0, zv1)
    pch = (p0, p1)
    gsem = (g0, g1)
    osem = (o0, o1)

    cid = lax.axis_index("c")
    sid = lax.axis_index("s")

    srow = sid * SRT
    pltpu.sync_copy(zs_hbm.at[cid].at[pl.ds(srow, SRT)], ztbl.at[pl.ds(srow, SRT)])
    plsc.subcore_barrier()

    npt = NCH // NS
    nsw = npt // SWEEP
    lane = lax.iota(jnp.int32, L)

    for sw in range(nsw):
        base = sid * npt + sw * SWEEP
        pltpu.async_copy(u_hbm.at[pl.ds(base, SWEEP)], uch2, isem)
        pltpu.async_copy(v_hbm.at[pl.ds(base, SWEEP)], vch2, isem)
        pltpu.make_async_copy(u_hbm.at[pl.ds(base, SWEEP)], uch2, isem).wait()
        pltpu.make_async_copy(v_hbm.at[pl.ds(base, SWEEP)], vch2, isem).wait()

        def issue_gather(g, b):
            pltpu.async_copy(ztbl.at[uch2.at[g]], zu[b], gsem[b])
            pltpu.async_copy(ztbl.at[vch2.at[g]], zv[b], gsem[b])

        def wait_gather(g, b):
            pltpu.make_async_copy(ztbl.at[uch2.at[g]], zu[b], gsem[b]).wait()
            pltpu.make_async_copy(ztbl.at[vch2.at[g]], zv[b], gsem[b]).wait()

        def issue_out(g, b, base=base):
            pltpu.async_copy(pch[b], out_hbm.at[cid].at[base + g], osem[b])

        def wait_out(g, b, base=base):
            pltpu.make_async_copy(pch[b], out_hbm.at[cid].at[base + g], osem[b]).wait()

        def compute(g, b):
            for j in range(CHUNK // L):

                def estep(e, vec):
                    row = j * L + e
                    parts = []
                    for kk in range(FH // L):
                        a = zu[b][row, pl.ds(kk * L, L)]
                        c = zv[b][row, pl.ds(kk * L, L)]
                        parts.append(a * c)
                    s = (parts[0] + parts[1]) + (parts[2] + parts[3])
                    d = jnp.sum(s)
                    return jnp.where(lane == e, d, vec)

                vec = lax.fori_loop(0, L, estep,
                                    jnp.zeros((L,), jnp.float32), unroll=4)
                pch[b][pl.ds(j * L, L)] = vec

        issue_gather(0, 0)

        @pl.loop(0, SWEEP // 2)
        def _(i):
            for b in (0, 1):
                g = i * 2 + b
                b2 = 1 - b

                @pl.when(g + 1 < SWEEP)
                def _():
                    issue_gather(g + 1, b2)

                wait_gather(g, b)

                @pl.when(g >= 2)
                def _():
                    wait_out(g - 2, b)

                compute(g, b)
                issue_out(g, b)

        wait_out(SWEEP - 2, 0)
        wait_out(SWEEP - 1, 1)


def _decode(zs, u_pad, v_pad):
    mesh = plsc.VectorSubcoreMesh(core_axis_name="c", subcore_axis_name="s")
    k = pl.kernel(
        _dec_body,
        name="scdec",
        out_type=jax.ShapeDtypeStruct((NC, NCH, CHUNK), jnp.float32),
        mesh=mesh,
        compiler_params=_sc_params(tc_tiling=False),
        scratch_types=(
            [pltpu.VMEM((SWEEP, CHUNK), jnp.int32)] * 2
            + [pltpu.VMEM((CHUNK, FH), jnp.float32)] * 4
            + [pltpu.VMEM((CHUNK,), jnp.float32)] * 2
            + [pltpu.VMEM_SHARED((N, FH), jnp.float32)]
            + [pltpu.SemaphoreType.DMA] * 5
        ),
    )
    return k(zs, u_pad, v_pad)


def _tc(fn, out_shape, *args):
    return pl.pallas_call(fn, out_shape=out_shape)(*args)


def _f32(shape):
    return jax.ShapeDtypeStruct(shape, jnp.float32)


def _cat(p_r):
    return jnp.concatenate([p_r[0, :N, :], p_r[1, :N, :]], axis=1)


def _tcsplit(x_r, s_r):
    x = x_r[...]
    s_r[0, :, :] = x[:, :FH]
    s_r[1, :, :] = x[:, FH:]


def _split(x):
    return _tc(_tcsplit, _f32((NC, N, FH)), x)


def _tc1(x_r, w1_r, w1s_r, h1_r, hs_r):
    x = x_r[...]
    h1_r[...] = _dot(x, w1_r[...])
    hs_r[...] = _dot(x, w1s_r[...])


def _tc2(h1_r, hp_r, h1p_r, dis_r, deg_r):
    degv = 1.0 + jnp.sum(hp_r[...][:, :N], axis=0)
    dis = lax.rsqrt(degv)[:, None]
    deg_r[...] = degv[:, None]
    dis_r[...] = dis
    h1p_r[...] = h1_r[...] * dis


def _tc3(p_r, h1p_r, dis_r, w2_r, b1_r, g2p_r):
    agg = _cat(p_r)
    dis = dis_r[...]
    x2 = jnp.maximum((agg + h1p_r[...]) * dis + b1_r[...], 0.0)
    g2p_r[...] = _dot(x2, w2_r[...]) * dis


def _tc4(p_r, g2p_r, dis_r, b2_r, z_r):
    agg = _cat(p_r)
    z_r[...] = (agg + g2p_r[...]) * dis_r[...] + b2_r[...]


def _tcmask(pd_r, cu_r, cv_r, dv_r, du_r, cnt_r):
    d = pd_r[0] + pd_r[1]
    row = lax.broadcasted_iota(jnp.int32, (NCH, CHUNK), 0)
    col = lax.broadcasted_iota(jnp.int32, (NCH, CHUNK), 1)
    gi = row * CHUNK + col
    sel = (d > 0.0) & (gi < E)
    dv_r[...] = jnp.where(sel, cv_r[...], GARB)
    du_r[...] = jnp.where(sel, cu_r[...], GARB)
    cnt_r[...] = jnp.sum(sel.astype(jnp.float32)).astype(jnp.int32).reshape(1, 1)


def _tc5(hp_r, deg_r, hs_r, hsp_r, diss_r):
    degs = deg_r[...] + jnp.sum(hp_r[...][:, :N], axis=0)[:, None]
    diss = lax.rsqrt(degs)
    diss_r[...] = diss
    hsp_r[...] = hs_r[...] * diss


def _tc6(p_r, hsp_r, diss_r, w2s_r, b1s_r, g4p_r):
    agg = _cat(p_r)
    diss = diss_r[...]
    h2 = jnp.maximum((agg + hsp_r[...]) * diss + b1s_r[...], 0.0)
    g4p_r[...] = _dot(h2, w2s_r[...]) * diss


def _tc7(p_r, g4p_r, diss_r, b2s_r, out_r):
    agg = p_r[0, :N, :] + p_r[1, :N, :]
    o = (agg + g4p_r[...]) * diss_r[...] + b2s_r[...]
    mx = jnp.max(o, axis=1, keepdims=True)
    ls = mx + jnp.log(jnp.sum(jnp.exp(o - mx), axis=1, keepdims=True))
    out_r[...] = o - ls


def kernel(x, masked_nodes, pos_edge_index, neg_edge_index, edge_index,
           W1, b1, W2, b2, W1s, b1s, W2s, b2s):
    del masked_nodes
    ei = edge_index.astype(jnp.int32)
    pe = pos_edge_index.astype(jnp.int32)
    ne = neg_edge_index.astype(jnp.int32)
    npad = EPAD - E
    zpad = jnp.zeros((npad,), jnp.int32)
    gpad = jnp.full((npad,), GARB, jnp.int32)
    es = jnp.concatenate([ei[0], zpad]).reshape(NCH, CHUNK)
    ed = jnp.concatenate([ei[1], gpad]).reshape(NCH, CHUNK)
    cu = jnp.concatenate([pe[0], ne[0], zpad]).reshape(NCH, CHUNK)
    cv = jnp.concatenate([pe[1], ne[1], zpad]).reshape(NCH, CHUNK)

    b1r = b1.reshape(1, F)
    b2r = b2.reshape(1, F)
    b1sr = b1s.reshape(1, F)
    b2sr = b2s.reshape(1, NCLS)

    histE = _hist([ed])
    h1, hs = _tc(_tc1, (_f32((N, F)), _f32((N, F))), x, W1, W1s)
    h1p, dis, deg = _tc(
        _tc2, (_f32((N, F)), _f32((N, 1)), _f32((N, 1))), h1, histE)

    p1 = _agg(_split(h1p), [(es, ed)])
    g2p = _tc(_tc3, _f32((N, F)), p1, h1p, dis, W2, b1r)

    p2 = _agg(_split(g2p), [(es, ed)])
    z = _tc(_tc4, _f32((N, F)), p2, g2p, dis, b2r)

    pdot = _decode(_split(z), cu, cv)
    dstv, dstu, cnt = _tc(
        _tcmask,
        (jax.ShapeDtypeStruct((NCH, CHUNK), jnp.int32),
         jax.ShapeDtypeStruct((NCH, CHUNK), jnp.int32),
         jax.ShapeDtypeStruct((1, 1), jnp.int32)),
        pdot, cu, cv)

    histM = _hist([dstv, dstu])
    hsp, diss = _tc(_tc5, (_f32((N, F)), _f32((N, 1))), histM, deg, hs)

    p3 = _agg(_split(hsp), [(es, ed), (cu, dstv), (cv, dstu)])
    g4p = _tc(_tc6, _f32((N, NCLS)), p3, hsp, diss, W2s, b1sr)

    p4 = _agg(g4p, [(es, ed), (cu, dstv), (cv, dstu)])
    logits = _tc(_tc7, _f32((N, NCLS)), p4, g4p, diss, b2sr)

    return (logits, z, cnt.reshape(()))


# speedup vs baseline: 1.9735x; 1.0041x over previous
import dataclasses

import jax
import jax.numpy as jnp
from jax import lax
from jax.experimental import pallas as pl
from jax.experimental.pallas import tpu as pltpu
from jax.experimental.pallas import tpu_sc as plsc

N = 10000
F = 128
FH = 64
NCLS = 16

NC = 2
NS = 16
L = 16
NW = NC * NS

GARB = N
NACC = N + 112
RPT = NACC // NS
SRT = N // NS

CHUNK = 128
SWEEP = 80
E = 320000
EPAD = 327680
NCH = EPAD // CHUNK

_HIGH = lax.Precision.HIGHEST


def _sc_params(tc_tiling=True):
    cp = pltpu.CompilerParams()
    if "needs_layout_passes" in pltpu.CompilerParams.__dataclass_fields__:
        cp = dataclasses.replace(cp, needs_layout_passes=False)
    if not tc_tiling:
        cp = dataclasses.replace(cp, use_tc_tiling_on_sc=False)
    return cp


def _dot(a, b):
    return lax.dot_general(a, b, (((1,), (0,)), ((), ())),
                           precision=_HIGH, preferred_element_type=jnp.float32)


def _zero_1d(ref, nwords):
    @pl.loop(0, nwords // L)
    def _(i):
        ref[pl.ds(i * L, L)] = jnp.zeros((L,), jnp.float32)


def _make_hist_body(nlists):
    def body(*refs):
        d_hbms = refs[:nlists]
        out_hbm = refs[nlists]
        dch, hacc, sem = refs[nlists + 1:]

        cid = lax.axis_index("c")
        sid = lax.axis_index("s")
        wid = sid * NC + cid

        _zero_1d(hacc, NACC)
        npw = NCH // NW
        ones = jnp.ones((L,), jnp.float32)

        for d_hbm in d_hbms:
            pltpu.async_copy(d_hbm.at[pl.ds(wid * npw, npw)], dch, sem).wait()

            @pl.loop(0, npw)
            def _(g):
                for j in range(CHUNK // L):
                    didv = dch[g, pl.ds(j * L, L)]
                    plsc.addupdate_scatter(hacc, [didv], ones)

        pltpu.sync_copy(hacc, out_hbm.at[wid])

    return body


def _hist(arrs):
    mesh = plsc.VectorSubcoreMesh(core_axis_name="c", subcore_axis_name="s")
    k = pl.kernel(
        _make_hist_body(len(arrs)),
        out_type=jax.ShapeDtypeStruct((NW, NACC), jnp.float32),
        mesh=mesh,
        name=f"schist{len(arrs)}",
        compiler_params=_sc_params(),
        scratch_types=[
            pltpu.VMEM((NCH // NW, CHUNK), jnp.int32),
            pltpu.VMEM((NACC,), jnp.float32),
            pltpu.SemaphoreType.DMA,
        ],
    )
    return k(*arrs)


def _make_agg_body(d, nchs, split_features):
    nl = len(nchs)
    ninp = 1

    def body(*refs):
        h_hbms = refs[:ninp]
        edges = refs[ninp:ninp + 2 * nl]
        out_hbm = refs[ninp + 2 * nl]
        (sidx2, didx2, r0, r1, tbl, acc, isem, a0, a1) = refs[ninp + 1 + 2 * nl:]
        rows = (r0, r1)
        ssem = (a0, a1)

        cid = lax.axis_index("c")
        sid = lax.axis_index("s")
        wid = sid * NC + cid

        srow = sid * SRT
        src_tbl = h_hbms[0].at[cid] if split_features else h_hbms[0]
        pltpu.sync_copy(src_tbl.at[pl.ds(srow, SRT)], tbl.at[pl.ds(srow, SRT)])

        @pl.loop(0, CHUNK)
        def _(r):
            @pl.loop(0, d // L if d >= L else 1)
            def _(k):
                r0[r, pl.ds(k * L, L)] = jnp.zeros((L,), jnp.float32)

        tb = sid * RPT

        @pl.loop(0, RPT // CHUNK)
        def _(i):
            pltpu.sync_copy(r0, acc.at[pl.ds(tb + i * CHUNK, CHUNK)])

        rem = RPT % CHUNK
        if rem:
            pltpu.sync_copy(r0.at[pl.ds(0, rem)],
                            acc.at[pl.ds(tb + (RPT // CHUNK) * CHUNK, rem)])
        def list_params(li):
            if split_features:
                return nchs[li] // NS, lax.axis_index("s") * (nchs[li] // NS)
            return nchs[li] // NW, wid * (nchs[li] // NW)

        _, pre_base = list_params(0)
        pltpu.async_copy(edges[0].at[pl.ds(pre_base, SWEEP)], sidx2, isem)
        pltpu.async_copy(edges[1].at[pl.ds(pre_base, SWEEP)], didx2, isem)

        plsc.subcore_barrier()

        for li in range(nl):
            s_hbm = edges[2 * li]
            dd_hbm = edges[2 * li + 1]
            npt, base0 = list_params(li)
            nsw = npt // SWEEP

            for sw in range(nsw):
                base = base0 + sw * SWEEP
                if not (li == 0 and sw == 0):
                    pltpu.async_copy(s_hbm.at[pl.ds(base, SWEEP)], sidx2, isem)
                    pltpu.async_copy(dd_hbm.at[pl.ds(base, SWEEP)], didx2, isem)
                pltpu.make_async_copy(s_hbm.at[pl.ds(base, SWEEP)], sidx2, isem).wait()
                pltpu.make_async_copy(dd_hbm.at[pl.ds(base, SWEEP)], didx2, isem).wait()

                def wait_scat(g, b):
                    pltpu.make_async_copy(rows[b], acc.at[didx2.at[g]], ssem[b]).wait()

                @pl.loop(0, SWEEP // 2)
                def _(i):
                    for b in (0, 1):
                        g = i * 2 + b

                        @pl.when(g >= 2)
                        def _():
                            wait_scat(g - 2, b)

                        gd = pltpu.async_copy(tbl.at[sidx2.at[g]], rows[b], ssem[b])
                        gd.wait()
                        pltpu.async_copy(rows[b], acc.at[didx2.at[g]], ssem[b], add=True)

                wait_scat(SWEEP - 2, 0)
                wait_scat(SWEEP - 1, 1)

        plsc.subcore_barrier()
        pltpu.sync_copy(acc.at[pl.ds(tb, RPT)], out_hbm.at[cid].at[pl.ds(tb, RPT)])

    return body


def _agg(h, pairs):
    split = h.ndim == 3
    dst = int(h.shape[-1])
    d = dst * (2 if split else 1)
    nchs = tuple(int(s.shape[0]) for s, _ in pairs)
    mesh = plsc.VectorSubcoreMesh(core_axis_name="c", subcore_axis_name="s")
    k = pl.kernel(
        _make_agg_body(dst, nchs, split),
        out_type=jax.ShapeDtypeStruct((NC, NACC, dst), jnp.float32),
        mesh=mesh,
        name=f"scagg{d}_{len(nchs)}",
        compiler_params=_sc_params(tc_tiling=False),
        scratch_types=[
            pltpu.VMEM((SWEEP, CHUNK), jnp.int32),
            pltpu.VMEM((SWEEP, CHUNK), jnp.int32),
            pltpu.VMEM((CHUNK, dst), jnp.float32),
            pltpu.VMEM((CHUNK, dst), jnp.float32),
            pltpu.VMEM_SHARED((N, dst), jnp.float32),
            pltpu.VMEM_SHARED((NACC, dst), jnp.float32),
            pltpu.SemaphoreType.DMA,
            pltpu.SemaphoreType.DMA,
            pltpu.SemaphoreType.DMA,
        ],
    )
    args = [h]
    for s, dd in pairs:
        args += [s, dd]
    return k(*args)


def _dec_body(zs_hbm, u_hbm, v_hbm, out_hbm,
              uch2, vch2, zu0, zu1, zv0, zv1, p0, p1, ztbl,
              isem, g0, g1, o0, o1):
    zu = (zu0, zu1)
    zv = (zv0, zv1)
    pch = (p0, p1)
    gsem = (g0, g1)
    osem = (o0, o1)

    cid = lax.axis_index("c")
    sid = lax.axis_index("s")

    srow = sid * SRT
    pltpu.sync_copy(zs_hbm.at[cid].at[pl.ds(srow, SRT)], ztbl.at[pl.ds(srow, SRT)])
    plsc.subcore_barrier()

    npt = NCH // NS
    nsw = npt // SWEEP
    lane = lax.iota(jnp.int32, L)

    for sw in range(nsw):
        base = sid * npt + sw * SWEEP
        pltpu.async_copy(u_hbm.at[pl.ds(base, SWEEP)], uch2, isem)
        pltpu.async_copy(v_hbm.at[pl.ds(base, SWEEP)], vch2, isem)
        pltpu.make_async_copy(u_hbm.at[pl.ds(base, SWEEP)], uch2, isem).wait()
        pltpu.make_async_copy(v_hbm.at[pl.ds(base, SWEEP)], vch2, isem).wait()

        def issue_gather(g, b):
            pltpu.async_copy(ztbl.at[uch2.at[g]], zu[b], gsem[b])
            pltpu.async_copy(ztbl.at[vch2.at[g]], zv[b], gsem[b])

        def wait_gather(g, b):
            pltpu.make_async_copy(ztbl.at[uch2.at[g]], zu[b], gsem[b]).wait()
            pltpu.make_async_copy(ztbl.at[vch2.at[g]], zv[b], gsem[b]).wait()

        def issue_out(g, b, base=base):
            pltpu.async_copy(pch[b], out_hbm.at[cid].at[base + g], osem[b])

        def wait_out(g, b, base=base):
            pltpu.make_async_copy(pch[b], out_hbm.at[cid].at[base + g], osem[b]).wait()

        def compute(g, b):
            for j in range(CHUNK // L):

                def estep(e, vec):
                    row = j * L + e
                    parts = []
                    for kk in range(FH // L):
                        a = zu[b][row, pl.ds(kk * L, L)]
                        c = zv[b][row, pl.ds(kk * L, L)]
                        parts.append(a * c)
                    s = (parts[0] + parts[1]) + (parts[2] + parts[3])
                    d = jnp.sum(s)
                    return jnp.where(lane == e, d, vec)

                vec = lax.fori_loop(0, L, estep,
                                    jnp.zeros((L,), jnp.float32), unroll=4)
                pch[b][pl.ds(j * L, L)] = vec

        issue_gather(0, 0)

        @pl.loop(0, SWEEP // 2)
        def _(i):
            for b in (0, 1):
                g = i * 2 + b
                b2 = 1 - b

                @pl.when(g + 1 < SWEEP)
                def _():
                    issue_gather(g + 1, b2)

                wait_gather(g, b)

                @pl.when(g >= 2)
                def _():
                    wait_out(g - 2, b)

                compute(g, b)
                issue_out(g, b)

        wait_out(SWEEP - 2, 0)
        wait_out(SWEEP - 1, 1)


def _decode(zs, u_pad, v_pad):
    mesh = plsc.VectorSubcoreMesh(core_axis_name="c", subcore_axis_name="s")
    k = pl.kernel(
        _dec_body,
        name="scdec",
        out_type=jax.ShapeDtypeStruct((NC, NCH, CHUNK), jnp.float32),
        mesh=mesh,
        compiler_params=_sc_params(tc_tiling=False),
        scratch_types=(
            [pltpu.VMEM((SWEEP, CHUNK), jnp.int32)] * 2
            + [pltpu.VMEM((CHUNK, FH), jnp.float32)] * 4
            + [pltpu.VMEM((CHUNK,), jnp.float32)] * 2
            + [pltpu.VMEM_SHARED((N, FH), jnp.float32)]
            + [pltpu.SemaphoreType.DMA] * 5
        ),
    )
    return k(zs, u_pad, v_pad)


def _tc(fn, out_shape, *args):
    return pl.pallas_call(fn, out_shape=out_shape)(*args)


def _f32(shape):
    return jax.ShapeDtypeStruct(shape, jnp.float32)


def _cat(p_r):
    return jnp.concatenate([p_r[0, :N, :], p_r[1, :N, :]], axis=1)


def _tc1(x_r, w1_r, w1s_r, h1_r, hs_r):
    x = x_r[...]
    h1_r[...] = _dot(x, w1_r[...])
    hs_r[...] = _dot(x, w1s_r[...])


def _tcsplit(x_r, s_r):
    x = x_r[...]
    s_r[0, :, :] = x[:, :FH]
    s_r[1, :, :] = x[:, FH:]


def _split(x):
    return _tc(_tcsplit, _f32((NC, N, FH)), x)


def _tc2(h1_r, hp_r, h1p_r, dis_r, deg_r):
    degv = 1.0 + jnp.sum(hp_r[...][:, :N], axis=0)
    dis = lax.rsqrt(degv)[:, None]
    deg_r[...] = degv[:, None]
    dis_r[...] = dis
    h1p_r[...] = h1_r[...] * dis


def _tc3(p_r, h1p_r, dis_r, w2_r, b1_r, g2p_r):
    agg = _cat(p_r)
    dis = dis_r[...]
    x2 = jnp.maximum((agg + h1p_r[...]) * dis + b1_r[...], 0.0)
    g2p_r[...] = _dot(x2, w2_r[...]) * dis


def _tc4(p_r, g2p_r, dis_r, b2_r, z_r):
    agg = _cat(p_r)
    z_r[...] = (agg + g2p_r[...]) * dis_r[...] + b2_r[...]


def _tcmask(pd_r, cu_r, cv_r, dv_r, du_r, cnt_r):
    d = pd_r[0] + pd_r[1]
    row = lax.broadcasted_iota(jnp.int32, (NCH, CHUNK), 0)
    col = lax.broadcasted_iota(jnp.int32, (NCH, CHUNK), 1)
    gi = row * CHUNK + col
    sel = (d > 0.0) & (gi < E)
    dv_r[...] = jnp.where(sel, cv_r[...], GARB)
    du_r[...] = jnp.where(sel, cu_r[...], GARB)
    cnt_r[...] = jnp.sum(sel.astype(jnp.float32)).astype(jnp.int32).reshape(1, 1)


def _tc5(hp_r, deg_r, hs_r, hsp_r, diss_r):
    degs = deg_r[...] + jnp.sum(hp_r[...][:, :N], axis=0)[:, None]
    diss = lax.rsqrt(degs)
    diss_r[...] = diss
    hsp_r[...] = hs_r[...] * diss


def _tc6(p_r, hsp_r, diss_r, w2s_r, b1s_r, g4p_r):
    agg = _cat(p_r)
    diss = diss_r[...]
    h2 = jnp.maximum((agg + hsp_r[...]) * diss + b1s_r[...], 0.0)
    g4p_r[...] = _dot(h2, w2s_r[...]) * diss


def _tc7(p_r, g4p_r, diss_r, b2s_r, out_r):
    agg = p_r[0, :N, :] + p_r[1, :N, :]
    o = (agg + g4p_r[...]) * diss_r[...] + b2s_r[...]
    mx = jnp.max(o, axis=1, keepdims=True)
    ls = mx + jnp.log(jnp.sum(jnp.exp(o - mx), axis=1, keepdims=True))
    out_r[...] = o - ls


def kernel(x, masked_nodes, pos_edge_index, neg_edge_index, edge_index,
           W1, b1, W2, b2, W1s, b1s, W2s, b2s):
    del masked_nodes
    ei = edge_index.astype(jnp.int32)
    pe = pos_edge_index.astype(jnp.int32)
    ne = neg_edge_index.astype(jnp.int32)
    npad = EPAD - E
    zpad = jnp.zeros((npad,), jnp.int32)
    gpad = jnp.full((npad,), GARB, jnp.int32)
    es = jnp.concatenate([ei[0], zpad]).reshape(NCH, CHUNK)
    ed = jnp.concatenate([ei[1], gpad]).reshape(NCH, CHUNK)
    cu = jnp.concatenate([pe[0], ne[0], zpad]).reshape(NCH, CHUNK)
    cv = jnp.concatenate([pe[1], ne[1], zpad]).reshape(NCH, CHUNK)

    b1r = b1.reshape(1, F)
    b2r = b2.reshape(1, F)
    b1sr = b1s.reshape(1, F)
    b2sr = b2s.reshape(1, NCLS)

    histE = _hist([ed])
    h1, hs = _tc(_tc1, (_f32((N, F)), _f32((N, F))), x, W1, W1s)
    h1p, dis, deg = _tc(
        _tc2, (_f32((N, F)), _f32((N, 1)), _f32((N, 1))), h1, histE)

    p1 = _agg(_split(h1p), [(es, ed)])
    g2p = _tc(_tc3, _f32((N, F)), p1, h1p, dis, W2, b1r)

    p2 = _agg(_split(g2p), [(es, ed)])
    z = _tc(_tc4, _f32((N, F)), p2, g2p, dis, b2r)

    pdot = _decode(_split(z), cu, cv)
    dstv, dstu, cnt = _tc(
        _tcmask,
        (jax.ShapeDtypeStruct((NCH, CHUNK), jnp.int32),
         jax.ShapeDtypeStruct((NCH, CHUNK), jnp.int32),
         jax.ShapeDtypeStruct((1, 1), jnp.int32)),
        pdot, cu, cv)

    histM = _hist([dstv, dstu])
    hsp, diss = _tc(_tc5, (_f32((N, F)), _f32((N, 1))), histM, deg, hs)

    p3 = _agg(_split(hsp), [(es, ed), (cu, dstv), (cv, dstu)])
    g4p = _tc(_tc6, _f32((N, NCLS)), p3, hsp, diss, W2s, b1sr)

    p4 = _agg(g4p, [(es, ed), (cu, dstv), (cv, dstu)])
    logits = _tc(_tc7, _f32((N, NCLS)), p4, g4p, diss, b2sr)

    return (logits, z, cnt.reshape(()))
